# Initial kernel scaffold; baseline (speedup 1.0000x reference)
import jax
import jax.numpy as jnp
from jax.experimental import pallas as pl


def kernel(embed, edge_index, W1, b1, W2, b2):
    raise NotImplementedError("write your pallas kernel here")


# speedup vs baseline: 3.3834x; 3.3834x over previous
import dataclasses
import functools

import jax
import jax.numpy as jnp
from jax import lax
from jax.experimental import pallas as pl
from jax.experimental.pallas import tpu as pltpu
from jax.experimental.pallas import tpu_sc as plsc

N = 10000
E = 320000
D = 128
H = 64

NC = 2
NS = 16
NW = NC * NS
EPW = E // NW
C = 80
NCHUNK = EPW // C
RPT = N // NS
L = 16
NPAD = N + 8
DUMP = N
MAXR = 4


def _proj_body(emb_ref, w1a_ref, w1b_ref, b1_ref, a_ref, b_ref):
    e = emb_ref[...]
    a_ref[...] = (
        jnp.dot(e, w1a_ref[...], preferred_element_type=jnp.float32) + b1_ref[...]
    )
    b_ref[...] = jnp.dot(e, w1b_ref[...], preferred_element_type=jnp.float32)


def _combine_body(p_ref, o_ref):
    o_ref[...] = p_ref[0] + p_ref[1]


def _sc_body(ap_hbm, bp_hbm, emb_hbm, src_hbm, dst_hbm, w2_hbm, b2_hbm,
             parts_hbm, ev_hbm,
             sidx, didx, av, bv, fv, evb, pbuf, w2t, b2t, zbuf, acc):
    c_id = lax.axis_index("c")
    s_id = lax.axis_index("s")
    wid = s_id * NC + c_id
    base_e = wid * EPW

    pltpu.sync_copy(w2_hbm, w2t)
    pltpu.sync_copy(b2_hbm, b2t)

    zvec = jnp.zeros((L,), jnp.float32)
    for r in range(zbuf.shape[0]):
        for j in range(D // L):
            zbuf[r, pl.ds(j * L, L)] = zvec
    nrch = N // 16

    @pl.loop(0, pl.cdiv(nrch, NS))
    def _(i):
        k = s_id + i * NS

        @pl.when(k < nrch)
        def _():
            pltpu.sync_copy(zbuf, acc.at[pl.ds(k * 16, 16)])

    plsc.subcore_barrier()

    iota = lax.iota(jnp.int32, L)

    def _chunk(ci, sl):
        didx_s = didx.at[sl]
        fv_s = fv.at[sl]
        evb_s = evb.at[sl]
        eb = base_e + ci * C
        pltpu.sync_copy(src_hbm.at[pl.ds(eb, C)], sidx)
        pltpu.sync_copy(dst_hbm.at[pl.ds(eb, C)], didx_s)
        pltpu.sync_copy(ap_hbm.at[sidx], av)
        pltpu.sync_copy(bp_hbm.at[didx_s], bv)
        pltpu.sync_copy(emb_hbm.at[sidx], fv_s)

        w2v = [w2t[pl.ds(L * j, L)] for j in range(H // L)]
        b2v = b2t[...]
        for g in range(C // L):
            e0 = g * L
            for l in range(L):
                accv = jnp.zeros((L,), jnp.float32)
                for j in range(H // L):
                    t = av[e0 + l, pl.ds(L * j, L)] + bv[e0 + l, pl.ds(L * j, L)]
                    accv = accv + jnp.maximum(t, 0.0) * w2v[j]
                pbuf[l, pl.ds(0, L)] = accv
            logit = b2v
            for j in range(L):
                logit = logit + plsc.load_gather(
                    pbuf, [iota, jnp.full((L,), j, jnp.int32)])
            evv = 1.0 / (1.0 + jnp.exp(-logit))
            evb_s[pl.ds(e0, L)] = evv
            for l in range(L):
                evl = evv.at[jnp.full((L,), l, jnp.int32)].get(
                    mode="promise_in_bounds")
                for j in range(D // L):
                    fv_s[e0 + l, pl.ds(L * j, L)] = (
                        fv_s[e0 + l, pl.ds(L * j, L)] * evl)

        pltpu.sync_copy(evb_s, ev_hbm.at[pl.ds(eb, C)])
        pltpu.sync_copy(fv_s, acc.at[didx_s], add=True)

    @pl.loop(0, NCHUNK - 1, step=2)
    def _(ci):
        _chunk(ci, 0)
        _chunk(ci + 1, 1)

    _chunk(NCHUNK - 1, 0)

    plsc.subcore_barrier()

    @pl.loop(0, pl.cdiv(N // 80, NS))
    def _(i):
        k = s_id + i * NS

        @pl.when(k < N // 80)
        def _():
            pltpu.sync_copy(acc.at[pl.ds(k * 80, 80)],
                            parts_hbm.at[c_id, pl.ds(k * 80, 80)])


def _sc_call(aprime, bproj, embed, src, dst, w2, b2b):
    cp = pltpu.CompilerParams(use_tc_tiling_on_sc=False)
    if "needs_layout_passes" in pltpu.CompilerParams.__dataclass_fields__:
        cp = dataclasses.replace(cp, needs_layout_passes=False)
    mesh = plsc.VectorSubcoreMesh(core_axis_name="c", subcore_axis_name="s")
    kern = pl.kernel(
        _sc_body,
        out_type=[
            jax.ShapeDtypeStruct((NC, N, D), jnp.float32),
            jax.ShapeDtypeStruct((E,), jnp.float32),
        ],
        mesh=mesh,
        compiler_params=cp,
        scratch_types=[
            pltpu.VMEM((C,), jnp.int32),
            pltpu.VMEM((2, C), jnp.int32),
            pltpu.VMEM((C, H), jnp.float32),
            pltpu.VMEM((C, H), jnp.float32),
            pltpu.VMEM((2, C, D), jnp.float32),
            pltpu.VMEM((2, C), jnp.float32),
            pltpu.VMEM((L, L), jnp.float32),
            pltpu.VMEM((H,), jnp.float32),
            pltpu.VMEM((L,), jnp.float32),
            pltpu.VMEM((L, D), jnp.float32),
            pltpu.VMEM_SHARED((NPAD, D), jnp.float32),
        ],
    )
    return kern(aprime, bproj, embed, src, dst, w2, b2b)


def kernel(embed, edge_index, W1, b1, W2, b2):
    w1a = W1[:D]
    w1b = W1[D:]
    src = edge_index[0]
    dst = edge_index[1]

    rb = 1000
    aprime, bproj = pl.pallas_call(
        _proj_body,
        grid=(N // rb,),
        in_specs=[
            pl.BlockSpec((rb, D), lambda i: (i, 0)),
            pl.BlockSpec((D, H), lambda i: (0, 0)),
            pl.BlockSpec((D, H), lambda i: (0, 0)),
            pl.BlockSpec((1, H), lambda i: (0, 0)),
        ],
        out_specs=[
            pl.BlockSpec((rb, H), lambda i: (i, 0)),
            pl.BlockSpec((rb, H), lambda i: (i, 0)),
        ],
        out_shape=[
            jax.ShapeDtypeStruct((N, H), jnp.float32),
            jax.ShapeDtypeStruct((N, H), jnp.float32),
        ],
    )(embed, w1a, w1b, b1.reshape(1, H))

    parts, ev = _sc_call(aprime, bproj, embed, src, dst, W2[:, 0],
                         jnp.broadcast_to(b2, (L,)).astype(jnp.float32))

    out = pl.pallas_call(
        _combine_body,
        grid=(N // rb,),
        in_specs=[pl.BlockSpec((NC, rb, D), lambda i: (0, i, 0))],
        out_specs=pl.BlockSpec((rb, D), lambda i: (i, 0)),
        out_shape=jax.ShapeDtypeStruct((N, D), jnp.float32),
    )(parts)

    return out, ev


# speedup vs baseline: 7.7476x; 2.2899x over previous
import dataclasses
import functools

import jax
import jax.numpy as jnp
from jax import lax
from jax.experimental import pallas as pl
from jax.experimental.pallas import tpu as pltpu
from jax.experimental.pallas import tpu_sc as plsc

N = 10000
E = 320000
D = 128
H = 64

NC = 2
NS = 16
NW = NC * NS
EPW = E // NW
C = 80
NCHUNK = EPW // C
RPT = N // NS
L = 16
NPAD = N + 8
DUMP = N
MAXR = 4


def _proj_body(emb_ref, w1a_ref, w1b_ref, b1_ref, a_ref, b_ref):
    e = emb_ref[...]
    a_ref[...] = (
        jnp.dot(e, w1a_ref[...], preferred_element_type=jnp.float32) + b1_ref[...]
    )
    b_ref[...] = jnp.dot(e, w1b_ref[...], preferred_element_type=jnp.float32)


def _combine_body(p_ref, o_ref):
    o_ref[...] = p_ref[0] + p_ref[1]


def _sc_body(ap_hbm, bp_hbm, emb_hbm, src_hbm, dst_hbm, w2_hbm, b2_hbm,
             parts_hbm, ev_hbm,
             sidx, didx, av, bv, fv, evb, pbuf, w2t, b2t, acc,
             sem_i0, sem_i1, sem_g0, sem_g1, sem_o0, sem_o1, sem_s0, sem_s1):
    c_id = lax.axis_index("c")
    s_id = lax.axis_index("s")
    wid = s_id * NC + c_id
    base_e = wid * EPW

    pltpu.sync_copy(w2_hbm, w2t)
    pltpu.sync_copy(b2_hbm, b2t)

    zvec = jnp.zeros((L,), jnp.float32)
    zbuf = fv.at[0].at[pl.ds(0, L)]
    for r in range(L):
        for j in range(D // L):
            zbuf[r, pl.ds(j * L, L)] = zvec
    nrch = N // 16

    @pl.loop(0, pl.cdiv(nrch, NS))
    def _(i):
        k = s_id + i * NS

        @pl.when(k < nrch)
        def _():
            pltpu.sync_copy(zbuf, acc.at[pl.ds(k * 16, 16)])

    plsc.subcore_barrier()

    iota = lax.iota(jnp.int32, L)
    sem_i = [sem_i0, sem_i1]
    sem_g = [sem_g0, sem_g1]
    sem_o = [sem_o0, sem_o1]
    sem_s = [sem_s0, sem_s1]

    def _idx_start(k, b4):
        eb = base_e + k * C
        s = sem_i[b4 % 2]
        pltpu.async_copy(src_hbm.at[pl.ds(eb, C)], sidx.at[b4 % 2], s)
        pltpu.async_copy(dst_hbm.at[pl.ds(eb, C)], didx.at[b4], s)

    def _idx_wait(k, b4):
        eb = base_e + k * C
        s = sem_i[b4 % 2]
        pltpu.make_async_copy(src_hbm.at[pl.ds(eb, C)], sidx.at[b4 % 2], s).wait()
        pltpu.make_async_copy(dst_hbm.at[pl.ds(eb, C)], didx.at[b4], s).wait()

    def _gather_start(sl, b4):
        s = sem_g[sl]
        pltpu.async_copy(ap_hbm.at[sidx.at[sl]], av.at[sl], s)
        pltpu.async_copy(bp_hbm.at[didx.at[b4]], bv.at[sl], s)
        pltpu.async_copy(emb_hbm.at[sidx.at[sl]], fv.at[sl], s)

    def _gather_wait(sl, b4):
        s = sem_g[sl]
        pltpu.make_async_copy(ap_hbm.at[sidx.at[sl]], av.at[sl], s).wait()
        pltpu.make_async_copy(bp_hbm.at[didx.at[b4]], bv.at[sl], s).wait()
        pltpu.make_async_copy(emb_hbm.at[sidx.at[sl]], fv.at[sl], s).wait()

    def _out_start(k, sl, b4):
        eb = base_e + k * C
        pltpu.async_copy(evb.at[sl], ev_hbm.at[pl.ds(eb, C)], sem_o[sl])
        pltpu.async_copy(fv.at[sl], acc.at[didx.at[b4]], sem_s[sl], add=True)

    def _out_wait(k, sl, b4):
        eb = base_e + k * C
        pltpu.make_async_copy(evb.at[sl], ev_hbm.at[pl.ds(eb, C)],
                              sem_o[sl]).wait()
        pltpu.make_async_copy(fv.at[sl], acc.at[didx.at[b4]],
                              sem_s[sl]).wait()

    def _compute(sl, sl3):
        fv_s = fv.at[sl]
        av_s = av.at[sl]
        bv_s = bv.at[sl]
        evb_s = evb.at[sl]
        w2v = [w2t[pl.ds(L * j, L)] for j in range(H // L)]
        b2v = b2t[...]

        @pl.loop(0, C // L)
        def _(g):
            e0 = g * L
            for l in range(L):
                accv = jnp.zeros((L,), jnp.float32)
                for j in range(H // L):
                    t = (av_s[e0 + l, pl.ds(L * j, L)]
                         + bv_s[e0 + l, pl.ds(L * j, L)])
                    accv = accv + jnp.maximum(t, 0.0) * w2v[j]
                pbuf[l, pl.ds(0, L)] = accv
            logit = b2v
            for j in range(L):
                logit = logit + plsc.load_gather(
                    pbuf, [iota, jnp.full((L,), j, jnp.int32)])
            evv = 1.0 / (1.0 + jnp.exp(-logit))
            evb_s[pl.ds(e0, L)] = evv
            for l in range(L):
                evl = evv.at[jnp.full((L,), l, jnp.int32)].get(
                    mode="promise_in_bounds")
                for j in range(D // L):
                    fv_s[e0 + l, pl.ds(L * j, L)] = (
                        fv_s[e0 + l, pl.ds(L * j, L)] * evl)

    def _chunk(k, b, has_prev=True, has_next1=True, has_next2=True):
        sl = b % 2
        if has_prev:
            _out_wait(k - 1, 1 - sl, (b - 1) % 4)
        if has_next1:
            _idx_wait(k + 1, (b + 1) % 4)
            _gather_start(1 - sl, (b + 1) % 4)
        _gather_wait(sl, b)
        if has_next2:
            _idx_start(k + 2, (b + 2) % 4)
        _compute(sl, b)
        _out_start(k, sl, b)

    _idx_start(0, 0)
    _idx_wait(0, 0)
    _gather_start(0, 0)
    _idx_start(1, 1)
    _chunk(0, 0, has_prev=False)

    @pl.loop(0, NCHUNK - 5, step=4)
    def _(ci):
        _chunk(ci + 1, 1)
        _chunk(ci + 2, 2)
        _chunk(ci + 3, 3)
        _chunk(ci + 4, 0)

    _chunk(NCHUNK - 4, 1)
    _chunk(NCHUNK - 3, 2)
    _chunk(NCHUNK - 2, 3, has_next2=False)
    _chunk(NCHUNK - 1, 0, has_next1=False, has_next2=False)
    _out_wait(NCHUNK - 1, 0, 0)

    plsc.subcore_barrier()

    @pl.loop(0, pl.cdiv(N // 80, NS))
    def _(i):
        k = s_id + i * NS

        @pl.when(k < N // 80)
        def _():
            pltpu.sync_copy(acc.at[pl.ds(k * 80, 80)],
                            parts_hbm.at[c_id, pl.ds(k * 80, 80)])


def _sc_call(aprime, bproj, embed, src, dst, w2, b2b):
    cp = pltpu.CompilerParams(use_tc_tiling_on_sc=False)
    if "needs_layout_passes" in pltpu.CompilerParams.__dataclass_fields__:
        cp = dataclasses.replace(cp, needs_layout_passes=False)
    mesh = plsc.VectorSubcoreMesh(core_axis_name="c", subcore_axis_name="s")
    kern = pl.kernel(
        _sc_body,
        out_type=[
            jax.ShapeDtypeStruct((NC, N, D), jnp.float32),
            jax.ShapeDtypeStruct((E,), jnp.float32),
        ],
        mesh=mesh,
        compiler_params=cp,
        scratch_types=[
            pltpu.VMEM((2, C), jnp.int32),
            pltpu.VMEM((4, C), jnp.int32),
            pltpu.VMEM((2, C, H), jnp.float32),
            pltpu.VMEM((2, C, H), jnp.float32),
            pltpu.VMEM((2, C, D), jnp.float32),
            pltpu.VMEM((2, C), jnp.float32),
            pltpu.VMEM((L, L), jnp.float32),
            pltpu.VMEM((H,), jnp.float32),
            pltpu.VMEM((L,), jnp.float32),
            pltpu.VMEM_SHARED((NPAD, D), jnp.float32),
            pltpu.SemaphoreType.DMA,
            pltpu.SemaphoreType.DMA,
            pltpu.SemaphoreType.DMA,
            pltpu.SemaphoreType.DMA,
            pltpu.SemaphoreType.DMA,
            pltpu.SemaphoreType.DMA,
            pltpu.SemaphoreType.DMA,
            pltpu.SemaphoreType.DMA,
        ],
    )
    return kern(aprime, bproj, embed, src, dst, w2, b2b)


def kernel(embed, edge_index, W1, b1, W2, b2):
    w1a = W1[:D]
    w1b = W1[D:]
    src = edge_index[0]
    dst = edge_index[1]

    rb = 1000
    aprime, bproj = pl.pallas_call(
        _proj_body,
        grid=(N // rb,),
        in_specs=[
            pl.BlockSpec((rb, D), lambda i: (i, 0)),
            pl.BlockSpec((D, H), lambda i: (0, 0)),
            pl.BlockSpec((D, H), lambda i: (0, 0)),
            pl.BlockSpec((1, H), lambda i: (0, 0)),
        ],
        out_specs=[
            pl.BlockSpec((rb, H), lambda i: (i, 0)),
            pl.BlockSpec((rb, H), lambda i: (i, 0)),
        ],
        out_shape=[
            jax.ShapeDtypeStruct((N, H), jnp.float32),
            jax.ShapeDtypeStruct((N, H), jnp.float32),
        ],
    )(embed, w1a, w1b, b1.reshape(1, H))

    parts, ev = _sc_call(aprime, bproj, embed, src, dst, W2[:, 0],
                         jnp.broadcast_to(b2, (L,)).astype(jnp.float32))

    out = pl.pallas_call(
        _combine_body,
        grid=(N // rb,),
        in_specs=[pl.BlockSpec((NC, rb, D), lambda i: (0, i, 0))],
        out_specs=pl.BlockSpec((rb, D), lambda i: (i, 0)),
        out_shape=jax.ShapeDtypeStruct((N, D), jnp.float32),
    )(parts)

    return out, ev
